# Initial kernel scaffold; baseline (speedup 1.0000x reference)
import jax
import jax.numpy as jnp
from jax.experimental import pallas as pl


def kernel(pred, target):
    raise NotImplementedError("write your pallas kernel here")


# speedup vs baseline: 23.2891x; 23.2891x over previous
import functools

import jax
import jax.numpy as jnp
from jax.experimental import pallas as pl
from jax.experimental.pallas import tpu as pltpu

_LANES = 512
_ROWS = 64
_PAR = 24


def _shift_cols(x, s):
  return jnp.concatenate([x[:, s:], x[:, :s]], axis=1)


def _shift_rows(x, s):
  return jnp.concatenate([x[s:, :], x[:s, :]], axis=0)


def _tile_contrib(v, wms):
  base = jnp.abs(v)
  x = v
  det_w = None
  for s, wm in zip((1, 2, 4), wms):
    rl = _shift_cols(x, s)
    cs = x + rl
    cd = x - rl
    css = _shift_rows(cs, s)
    cds = _shift_rows(cd, s)
    det = (jnp.abs(cs - css)
           + jnp.abs(cd + cds)
           + jnp.abs(cd - cds))
    det_w = det * wm if det_w is None else det_w + det * wm
    if s < 4:
      x = cs + css
  return base, det_w


def _wavelet_kernel(inv_n, p_ref, t_ref, o_ref):
  j = pl.program_id(1)

  row = jax.lax.broadcasted_iota(jnp.int32, (8, 128), 0)
  col = jax.lax.broadcasted_iota(jnp.int32, (8, 128), 1)
  wms = []
  for k in (1, 2, 3):
    m = (1 << k) - 1
    mask = ((row & m) == 0) & ((col & m) == 0)
    wms.append(jnp.where(mask, jnp.float32((2.0 ** k) * inv_n),
                         jnp.float32(0.0)))

  e = p_ref[...] - t_ref[...]
  acc0 = None
  accw = None
  for r in range(_ROWS // 8):
    for c in range(_LANES // 128):
      v = e[r * 8:(r + 1) * 8, c * 128:(c + 1) * 128]
      b, dw = _tile_contrib(v, wms)
      acc0 = b if acc0 is None else acc0 + b
      accw = dw if accw is None else accw + dw
  part = accw + acc0 * jnp.float32(inv_n)

  @pl.when(j == 0)
  def _():
    o_ref[0] = part

  @pl.when(j != 0)
  def _():
    o_ref[0] = o_ref[0] + part


@jax.jit
def kernel(pred, target):
  n_total = pred.size
  p = pred.reshape(-1, _LANES)
  t = target.reshape(-1, _LANES)
  n_blocks = p.shape[0] // _ROWS
  par = _PAR if n_blocks % _PAR == 0 else 1
  n_inner = n_blocks // par

  body = functools.partial(_wavelet_kernel, 1.0 / n_total)
  out = pl.pallas_call(
      body,
      grid=(par, n_inner),
      in_specs=[
          pl.BlockSpec((_ROWS, _LANES), lambda i, j, ni=n_inner: (i * ni + j, 0)),
          pl.BlockSpec((_ROWS, _LANES), lambda i, j, ni=n_inner: (i * ni + j, 0)),
      ],
      out_specs=pl.BlockSpec((1, 8, 128), lambda i, j: (i, 0, 0)),
      out_shape=jax.ShapeDtypeStruct((par, 8, 128), jnp.float32),
      compiler_params=pltpu.CompilerParams(
          dimension_semantics=("parallel", "arbitrary"),
      ),
  )(p, t)
  return jnp.sum(out)


# speedup vs baseline: 52.3287x; 2.2469x over previous
import functools

import jax
import jax.numpy as jnp
from jax.experimental import pallas as pl
from jax.experimental.pallas import tpu as pltpu

_LANES = 512
_BLOCK_ROWS = 2048
_CHUNK = 64
_PAR = 24


def _shift_cols(x, s):
  return jnp.concatenate([x[:, s:], x[:, :s]], axis=1)


def _shift_rows(x, s):
  return jnp.concatenate([x[s:, :], x[:s, :]], axis=0)


def _tile_contrib(v, wms):
  base = jnp.abs(v)
  x = v
  det_w = None
  for s, wm in zip((1, 2, 4), wms):
    rl = _shift_cols(x, s)
    cs = x + rl
    cd = x - rl
    css = _shift_rows(cs, s)
    cds = _shift_rows(cd, s)
    det = (jnp.abs(cs - css)
           + jnp.abs(cd + cds)
           + jnp.abs(cd - cds))
    det_w = det * wm if det_w is None else det_w + det * wm
    if s < 4:
      x = cs + css
  return base, det_w


def _wavelet_kernel(inv_n, p_ref, t_ref, o_ref):
  row = jax.lax.broadcasted_iota(jnp.int32, (8, 128), 0)
  col = jax.lax.broadcasted_iota(jnp.int32, (8, 128), 1)
  wms = []
  for k in (1, 2, 3):
    m = (1 << k) - 1
    mask = ((row & m) == 0) & ((col & m) == 0)
    wms.append(jnp.where(mask, jnp.float32((2.0 ** k) * inv_n),
                         jnp.float32(0.0)))

  def body(it, carry):
    acc0, accw = carry
    base_row = it * _CHUNK
    for r in range(_CHUNK // 8):
      for c in range(_LANES // 128):
        rows = pl.ds(base_row + r * 8, 8)
        cols = slice(c * 128, (c + 1) * 128)
        v = p_ref[rows, cols] - t_ref[rows, cols]
        b, dw = _tile_contrib(v, wms)
        acc0 = acc0 + b
        accw = accw + dw
    return acc0, accw

  zero = jnp.zeros((8, 128), jnp.float32)
  acc0, accw = jax.lax.fori_loop(0, _BLOCK_ROWS // _CHUNK, body, (zero, zero))
  o_ref[0] = accw + acc0 * jnp.float32(inv_n)


@jax.jit
def kernel(pred, target):
  n_total = pred.size
  p = pred.reshape(-1, _LANES)
  t = target.reshape(-1, _LANES)
  par = p.shape[0] // _BLOCK_ROWS

  body = functools.partial(_wavelet_kernel, 1.0 / n_total)
  out = pl.pallas_call(
      body,
      grid=(par,),
      in_specs=[
          pl.BlockSpec((_BLOCK_ROWS, _LANES), lambda i: (i, 0)),
          pl.BlockSpec((_BLOCK_ROWS, _LANES), lambda i: (i, 0)),
      ],
      out_specs=pl.BlockSpec((1, 8, 128), lambda i: (i, 0, 0)),
      out_shape=jax.ShapeDtypeStruct((par, 8, 128), jnp.float32),
      compiler_params=pltpu.CompilerParams(
          dimension_semantics=("parallel",),
      ),
  )(p, t)
  return jnp.sum(out)
